# baseline (speedup 1.0000x reference)
import functools

import jax
import jax.numpy as jnp
from jax.experimental import pallas as pl
from jax.experimental.pallas import tpu as pltpu

_BLOCK = 2000


def _dage_kernel(nb_ref, cur_ref, rm_ref,
                 wn1_ref, wn2_ref, bn_ref,
                 wr1_ref, wr2_ref, br_ref,
                 wd1_ref, wd2_ref, bd_ref,
                 out_ref):
    cur = cur_ref[...]
    h_n = jnp.dot(nb_ref[...], wn1_ref[...], preferred_element_type=jnp.float32)
    h_n = h_n + jnp.dot(cur, wn2_ref[...], preferred_element_type=jnp.float32)
    h_n = jnp.maximum(h_n + bn_ref[...], 0.0)
    h_r = jnp.dot(rm_ref[...], wr1_ref[...], preferred_element_type=jnp.float32)
    h_r = h_r + jnp.dot(cur, wr2_ref[...], preferred_element_type=jnp.float32)
    h_r = jnp.maximum(h_r + br_ref[...], 0.0)
    out = jnp.dot(h_n, wd1_ref[...], preferred_element_type=jnp.float32)
    out = out + jnp.dot(h_r, wd2_ref[...], preferred_element_type=jnp.float32)
    out_ref[...] = out + bd_ref[...]


@jax.jit
def kernel(neighbor, current, remote, W_n, b_n, W_r, b_r, W_d, b_d):
    n, emb = neighbor.shape
    half = W_n.shape[1]
    dout = W_d.shape[1]
    grid = n // _BLOCK

    row_spec = pl.BlockSpec((_BLOCK, emb), lambda i: (i, 0))
    full = lambda shape: pl.BlockSpec(shape, lambda i: (0, 0))

    return pl.pallas_call(
        _dage_kernel,
        grid=(grid,),
        in_specs=[
            row_spec, row_spec, row_spec,
            full((emb, half)), full((emb, half)), full((1, half)),
            full((emb, half)), full((emb, half)), full((1, half)),
            full((half, dout)), full((half, dout)), full((1, dout)),
        ],
        out_specs=pl.BlockSpec((_BLOCK, dout), lambda i: (i, 0)),
        out_shape=jax.ShapeDtypeStruct((n, dout), jnp.float32),
        compiler_params=pltpu.CompilerParams(
            dimension_semantics=("arbitrary",),
        ),
    )(
        neighbor, current, remote,
        W_n[:emb], W_n[emb:], b_n.reshape(1, half),
        W_r[:emb], W_r[emb:], b_r.reshape(1, half),
        W_d[:half], W_d[half:], b_d.reshape(1, dout),
    )


# speedup vs baseline: 1.1061x; 1.1061x over previous
import functools

import jax
import jax.numpy as jnp
from jax.experimental import pallas as pl
from jax.experimental.pallas import tpu as pltpu

_BLOCK = 4000


def _dage_kernel(nb_ref, cur_ref, rm_ref,
                 wn1_ref, wn2_ref, bn_ref,
                 wr1_ref, wr2_ref, br_ref,
                 wd1_ref, wd2_ref, bd_ref,
                 out_ref):
    cur = cur_ref[...]
    h_n = jnp.dot(nb_ref[...], wn1_ref[...], preferred_element_type=jnp.float32)
    h_n = h_n + jnp.dot(cur, wn2_ref[...], preferred_element_type=jnp.float32)
    h_n = jnp.maximum(h_n + bn_ref[...], 0.0)
    h_r = jnp.dot(rm_ref[...], wr1_ref[...], preferred_element_type=jnp.float32)
    h_r = h_r + jnp.dot(cur, wr2_ref[...], preferred_element_type=jnp.float32)
    h_r = jnp.maximum(h_r + br_ref[...], 0.0)
    out = jnp.dot(h_n, wd1_ref[...], preferred_element_type=jnp.float32)
    out = out + jnp.dot(h_r, wd2_ref[...], preferred_element_type=jnp.float32)
    out_ref[...] = out + bd_ref[...]


@jax.jit
def kernel(neighbor, current, remote, W_n, b_n, W_r, b_r, W_d, b_d):
    n, emb = neighbor.shape
    half = W_n.shape[1]
    dout = W_d.shape[1]
    grid = n // _BLOCK

    row_spec = pl.BlockSpec((_BLOCK, emb), lambda i: (i, 0))
    full = lambda shape: pl.BlockSpec(shape, lambda i: (0, 0))

    return pl.pallas_call(
        _dage_kernel,
        grid=(grid,),
        in_specs=[
            row_spec, row_spec, row_spec,
            full((emb, half)), full((emb, half)), full((1, half)),
            full((emb, half)), full((emb, half)), full((1, half)),
            full((half, dout)), full((half, dout)), full((1, dout)),
        ],
        out_specs=pl.BlockSpec((_BLOCK, dout), lambda i: (i, 0)),
        out_shape=jax.ShapeDtypeStruct((n, dout), jnp.float32),
        compiler_params=pltpu.CompilerParams(
            dimension_semantics=("arbitrary",),
        ),
    )(
        neighbor, current, remote,
        W_n[:emb], W_n[emb:], b_n.reshape(1, half),
        W_r[:emb], W_r[emb:], b_r.reshape(1, half),
        W_d[:half], W_d[half:], b_d.reshape(1, dout),
    )


# speedup vs baseline: 1.1146x; 1.0077x over previous
import functools

import jax
import jax.numpy as jnp
from jax.experimental import pallas as pl
from jax.experimental.pallas import tpu as pltpu

_BLOCK = 5000


def _dage_kernel(nb_ref, cur_ref, rm_ref,
                 wn1_ref, wn2_ref, bn_ref,
                 wr1_ref, wr2_ref, br_ref,
                 wd1_ref, wd2_ref, bd_ref,
                 out_ref):
    cur = cur_ref[...]
    h_n = jnp.dot(nb_ref[...], wn1_ref[...], preferred_element_type=jnp.float32)
    h_n = h_n + jnp.dot(cur, wn2_ref[...], preferred_element_type=jnp.float32)
    h_n = jnp.maximum(h_n + bn_ref[...], 0.0)
    h_r = jnp.dot(rm_ref[...], wr1_ref[...], preferred_element_type=jnp.float32)
    h_r = h_r + jnp.dot(cur, wr2_ref[...], preferred_element_type=jnp.float32)
    h_r = jnp.maximum(h_r + br_ref[...], 0.0)
    out = jnp.dot(h_n, wd1_ref[...], preferred_element_type=jnp.float32)
    out = out + jnp.dot(h_r, wd2_ref[...], preferred_element_type=jnp.float32)
    out_ref[...] = out + bd_ref[...]


@jax.jit
def kernel(neighbor, current, remote, W_n, b_n, W_r, b_r, W_d, b_d):
    n, emb = neighbor.shape
    half = W_n.shape[1]
    dout = W_d.shape[1]
    grid = n // _BLOCK

    row_spec = pl.BlockSpec((_BLOCK, emb), lambda i: (i, 0))
    full = lambda shape: pl.BlockSpec(shape, lambda i: (0, 0))

    return pl.pallas_call(
        _dage_kernel,
        grid=(grid,),
        in_specs=[
            row_spec, row_spec, row_spec,
            full((emb, half)), full((emb, half)), full((1, half)),
            full((emb, half)), full((emb, half)), full((1, half)),
            full((half, dout)), full((half, dout)), full((1, dout)),
        ],
        out_specs=pl.BlockSpec((_BLOCK, dout), lambda i: (i, 0)),
        out_shape=jax.ShapeDtypeStruct((n, dout), jnp.float32),
        compiler_params=pltpu.CompilerParams(
            dimension_semantics=("arbitrary",),
        ),
    )(
        neighbor, current, remote,
        W_n[:emb], W_n[emb:], b_n.reshape(1, half),
        W_r[:emb], W_r[emb:], b_r.reshape(1, half),
        W_d[:half], W_d[half:], b_d.reshape(1, dout),
    )
